# baseline (speedup 1.0000x reference)
import functools

import jax
import jax.numpy as jnp
from jax import lax
from jax.experimental import pallas as pl
from jax.experimental.pallas import tpu as pltpu
from jax.experimental.pallas import tpu_sc as plsc

EMBED = 64
NUM_WORKERS = 32
CHUNK = 128
BLOCKS = 8
GROUP = 8
L = 16
JGROUPS = CHUNK // L


def _gather_body(table_hbm, idx3_hbm, out_hbm, idx_v, pc0, pc1, pb0, pb1,
                 oc0, oc1, gsem0, gsem1, wsem0, wsem1):
    c = lax.axis_index("c")
    s = lax.axis_index("s")
    wid = s * 2 + c
    ngroups, _, batch = idx3_hbm.shape
    seq = ngroups * GROUP
    base_rows = seq // NUM_WORKERS
    n_extra = seq - base_rows * NUM_WORKERS
    start = jnp.where(wid < n_extra, wid * (base_rows + 1),
                      n_extra * (base_rows + 1) + (wid - n_extra) * base_rows)
    n_rows = jnp.where(wid < n_extra, base_rows + 1, base_rows)
    g0 = start // GROUP
    lr_base = start - g0 * GROUP

    pltpu.sync_copy(idx3_hbm.at[g0], idx_v.at[pl.ds(0, GROUP)])

    @pl.when(g0 + 1 < ngroups)
    def _():
        pltpu.sync_copy(idx3_hbm.at[g0 + 1], idx_v.at[pl.ds(GROUP, GROUP)])

    def fire(pb, pc, sem, si, bb):
        lr = lr_base + si
        for j in range(JGROUPS):
            pc[pl.ds(j * L, L)] = idx_v[lr, pl.ds(bb * CHUNK + j * L, L)] >> 1
        pltpu.async_copy(table_hbm.at[pc], pb, sem)

    def drain(pb, sem):
        pltpu.make_async_copy(table_hbm.at[pc0], pb, sem).wait()

    def extract(pb, oc, si, bb):
        lr = lr_base + si

        def g_body(g, carry):
            vec = idx_v[lr, pl.ds(bb * CHUNK + g * L, L)]
            for k2 in range(L):
                off = (vec[k2] & 1) << 6
                j = g * L + k2
                orow = g * (L // 2) + k2 // 2
                obase = (k2 % 2) * EMBED
                for k in range(EMBED // L):
                    oc[orow, pl.ds(obase + k * L, L)] = (
                        pb[j, pl.ds(off + k * L, L)])
            return carry

        lax.fori_loop(0, JGROUPS, g_body, 0)

    def fire_wb(oc, row, bb, wsem):
        pltpu.async_copy(oc, out_hbm.at[row, pl.ds(bb * EMBED, EMBED)], wsem)

    def wait_wb(wsem):
        pltpu.make_async_copy(oc0, out_hbm.at[0, pl.ds(0, EMBED)], wsem).wait()

    fire(pb0, pc0, gsem0, 0, 0)

    def row_body(si, carry):
        row = start + si

        def t_body(t, c2):
            bb0 = 2 * t
            not_first = (si > 0) | (t > 0)
            drain(pb0, gsem0)
            fire(pb1, pc1, gsem1, si, bb0 + 1)

            @pl.when(not_first)
            def _():
                wait_wb(wsem0)

            extract(pb0, oc0, si, bb0)
            fire_wb(oc0, row, bb0, wsem0)

            drain(pb1, gsem1)
            nsi = jnp.where(t < 3, si, si + 1)
            nbb = (bb0 + 2) % BLOCKS

            @pl.when((t < 3) | (si < n_rows - 1))
            def _():
                fire(pb0, pc0, gsem0, nsi, nbb)

            @pl.when(not_first)
            def _():
                wait_wb(wsem1)

            extract(pb1, oc1, si, bb0 + 1)
            fire_wb(oc1, row, bb0 + 1, wsem1)
            return c2

        lax.fori_loop(0, BLOCKS // 2, t_body, 0)
        return carry

    lax.fori_loop(0, n_rows, row_body, 0)
    wait_wb(wsem0)
    wait_wb(wsem1)


def _make_gather(seq, batch):
    return functools.partial(
        pl.kernel,
        out_type=jax.ShapeDtypeStruct((seq, batch // 2, CHUNK), jnp.float32),
        mesh=plsc.VectorSubcoreMesh(core_axis_name="c", subcore_axis_name="s"),
        scratch_types=[
            pltpu.VMEM((2 * GROUP, batch), jnp.int32),
            pltpu.VMEM((CHUNK,), jnp.int32),
            pltpu.VMEM((CHUNK,), jnp.int32),
            pltpu.VMEM((CHUNK, CHUNK), jnp.float32),
            pltpu.VMEM((CHUNK, CHUNK), jnp.float32),
            pltpu.VMEM((EMBED, CHUNK), jnp.float32),
            pltpu.VMEM((EMBED, CHUNK), jnp.float32),
            pltpu.SemaphoreType.DMA,
            pltpu.SemaphoreType.DMA,
            pltpu.SemaphoreType.DMA,
            pltpu.SemaphoreType.DMA,
        ],
        compiler_params=pltpu.CompilerParams(use_tc_tiling_on_sc=True),
    )(_gather_body)


def kernel(word_input, character_input, word_embed):
    batch, seq = word_input.shape
    vocab = word_embed.shape[0]
    tableP = jnp.concatenate([word_embed[0::2], word_embed[1::2]], axis=1)
    idx3 = jnp.reshape(word_input.T, (seq // GROUP, GROUP, batch))
    out = _make_gather(seq, batch)(tableP, idx3)
    return jnp.transpose(jnp.reshape(out, (seq, batch, EMBED)), (1, 0, 2))


# speedup vs baseline: 6.1241x; 6.1241x over previous
import functools

import jax
import jax.numpy as jnp
from jax import lax
from jax.experimental import pallas as pl
from jax.experimental.pallas import tpu as pltpu
from jax.experimental.pallas import tpu_sc as plsc

EMBED = 64
NUM_WORKERS = 32
CHUNK = 128
BLOCKS = 8
GROUP = 8
L = 16
JGROUPS = CHUNK // L


def _gather_body(table_hbm, idx3_hbm, out_hbm, idx_v, pc0, pc1, pb0, pb1,
                 oc0, oc1, gsem0, gsem1, wsem0, wsem1):
    c = lax.axis_index("c")
    s = lax.axis_index("s")
    wid = s * 2 + c
    ngroups, _, batch = idx3_hbm.shape
    seq = ngroups * GROUP
    base_rows = seq // NUM_WORKERS
    n_extra = seq - base_rows * NUM_WORKERS
    start = jnp.where(wid < n_extra, wid * (base_rows + 1),
                      n_extra * (base_rows + 1) + (wid - n_extra) * base_rows)
    n_rows = jnp.where(wid < n_extra, base_rows + 1, base_rows)
    g0 = start // GROUP
    lr_base = start - g0 * GROUP

    pltpu.sync_copy(idx3_hbm.at[g0], idx_v.at[pl.ds(0, GROUP)])

    @pl.when(g0 + 1 < ngroups)
    def _():
        pltpu.sync_copy(idx3_hbm.at[g0 + 1], idx_v.at[pl.ds(GROUP, GROUP)])

    def fire(pb, pc, sem, si, bb):
        lr = lr_base + si
        for j in range(JGROUPS):
            pc[pl.ds(j * L, L)] = idx_v[lr, pl.ds(bb * CHUNK + j * L, L)] >> 1
        pltpu.async_copy(table_hbm.at[pc], pb, sem)

    def drain(pb, sem):
        pltpu.make_async_copy(table_hbm.at[pc0], pb, sem).wait()

    def extract(pb, oc, si, bb):
        lr = lr_base + si

        def g_body(g, carry):
            vec = idx_v[lr, pl.ds(bb * CHUNK + g * L, L)]
            for k2 in range(L):
                off = (vec[k2] & 1) << 6
                j = g * L + k2
                orow = g * (L // 2) + k2 // 2
                obase = (k2 % 2) * EMBED
                for k in range(EMBED // L):
                    oc[orow, pl.ds(obase + k * L, L)] = (
                        pb[j, pl.ds(off + k * L, L)])
            return carry

        lax.fori_loop(0, JGROUPS, g_body, 0)

    def fire_wb(oc, row, bb, wsem):
        pltpu.async_copy(oc, out_hbm.at[row, pl.ds(bb * EMBED, EMBED)], wsem)

    def wait_wb(wsem):
        pltpu.make_async_copy(oc0, out_hbm.at[0, pl.ds(0, EMBED)], wsem).wait()

    fire(pb0, pc0, gsem0, 0, 0)

    def row_body(si, carry):
        row = start + si

        def t_body(t, c2):
            bb0 = 2 * t
            not_first = (si > 0) | (t > 0)
            drain(pb0, gsem0)
            fire(pb1, pc1, gsem1, si, bb0 + 1)

            @pl.when(not_first)
            def _():
                wait_wb(wsem0)

            extract(pb0, oc0, si, bb0)
            fire_wb(oc0, row, bb0, wsem0)

            drain(pb1, gsem1)
            nsi = jnp.where(t < 3, si, si + 1)
            nbb = (bb0 + 2) % BLOCKS

            @pl.when((t < 3) | (si < n_rows - 1))
            def _():
                fire(pb0, pc0, gsem0, nsi, nbb)

            @pl.when(not_first)
            def _():
                wait_wb(wsem1)

            extract(pb1, oc1, si, bb0 + 1)
            fire_wb(oc1, row, bb0 + 1, wsem1)
            return c2

        lax.fori_loop(0, BLOCKS // 2, t_body, 0)
        return carry

    lax.fori_loop(0, n_rows, row_body, 0)
    wait_wb(wsem0)
    wait_wb(wsem1)


def _make_gather(seq, batch):
    return functools.partial(
        pl.kernel,
        out_type=jax.ShapeDtypeStruct((seq, batch // 2, CHUNK), jnp.float32),
        mesh=plsc.VectorSubcoreMesh(core_axis_name="c", subcore_axis_name="s"),
        scratch_types=[
            pltpu.VMEM((2 * GROUP, batch), jnp.int32),
            pltpu.VMEM((CHUNK,), jnp.int32),
            pltpu.VMEM((CHUNK,), jnp.int32),
            pltpu.VMEM((CHUNK, CHUNK), jnp.float32),
            pltpu.VMEM((CHUNK, CHUNK), jnp.float32),
            pltpu.VMEM((EMBED, CHUNK), jnp.float32),
            pltpu.VMEM((EMBED, CHUNK), jnp.float32),
            pltpu.SemaphoreType.DMA,
            pltpu.SemaphoreType.DMA,
            pltpu.SemaphoreType.DMA,
            pltpu.SemaphoreType.DMA,
        ],
        compiler_params=pltpu.CompilerParams(use_tc_tiling_on_sc=True),
    )(_gather_body)


def _pair_body(x_ref, o_ref):
    xT = jnp.transpose(x_ref[...], (1, 0))
    r3 = jnp.reshape(xT, (xT.shape[0] // 2, 2, xT.shape[1]))
    o_ref[...] = jnp.concatenate([r3[:, 0, :], r3[:, 1, :]], axis=1)


def _pair_table(word_embed):
    vocab, embed = word_embed.shape
    vb = 512
    grid = (vocab + vb - 1) // vb
    return pl.pallas_call(
        _pair_body,
        grid=(grid,),
        in_specs=[pl.BlockSpec((embed, vb), lambda i: (0, i))],
        out_specs=pl.BlockSpec((vb // 2, 2 * embed), lambda i: (i, 0)),
        out_shape=jax.ShapeDtypeStruct((vocab // 2, 2 * embed), jnp.float32),
    )(word_embed.T)


def kernel(word_input, character_input, word_embed):
    batch, seq = word_input.shape
    vocab = word_embed.shape[0]
    tableP = _pair_table(word_embed)
    idx3 = jnp.reshape(word_input.T, (seq // GROUP, GROUP, batch))
    out = _make_gather(seq, batch)(tableP, idx3)
    return jnp.transpose(jnp.reshape(out, (seq, batch, EMBED)), (1, 0, 2))


# speedup vs baseline: 11.2777x; 1.8415x over previous
import functools

import jax
import jax.numpy as jnp
from jax import lax
from jax.experimental import pallas as pl
from jax.experimental.pallas import tpu as pltpu
from jax.experimental.pallas import tpu_sc as plsc

EMBED = 64
NUM_WORKERS = 32
CHUNK = 128
BLOCKS = 8
GROUP = 8
L = 16
JGROUPS = CHUNK // L


def _gather_body(table_hbm, idx3_hbm, out_hbm, idx_v, pc0, pc1, pb0, pb1,
                 oc0, oc1, gsem0, gsem1, wsem0, wsem1):
    c = lax.axis_index("c")
    s = lax.axis_index("s")
    wid = s * 2 + c
    ngroups, _, batch = idx3_hbm.shape
    seq = ngroups * GROUP
    base_rows = seq // NUM_WORKERS
    n_extra = seq - base_rows * NUM_WORKERS
    start = jnp.where(wid < n_extra, wid * (base_rows + 1),
                      n_extra * (base_rows + 1) + (wid - n_extra) * base_rows)
    n_rows = jnp.where(wid < n_extra, base_rows + 1, base_rows)
    g0 = start // GROUP
    lr_base = start - g0 * GROUP

    pltpu.sync_copy(idx3_hbm.at[g0], idx_v.at[pl.ds(0, GROUP)])

    @pl.when(g0 + 1 < ngroups)
    def _():
        pltpu.sync_copy(idx3_hbm.at[g0 + 1], idx_v.at[pl.ds(GROUP, GROUP)])

    def fire(pb, pc, sem, si, bb):
        lr = lr_base + si
        for j in range(JGROUPS):
            pc[pl.ds(j * L, L)] = idx_v[lr, pl.ds(bb * CHUNK + j * L, L)] >> 1
        pltpu.async_copy(table_hbm.at[pc], pb, sem)

    def drain(pb, sem):
        pltpu.make_async_copy(table_hbm.at[pc0], pb, sem).wait()

    def extract(pb, oc, si, bb):
        lr = lr_base + si

        def g_body(g, carry):
            vec = idx_v[lr, pl.ds(bb * CHUNK + g * L, L)]
            for k2 in range(L):
                off = (vec[k2] & 1) << 6
                j = g * L + k2
                orow = g * (L // 2) + k2 // 2
                obase = (k2 % 2) * EMBED
                for k in range(EMBED // L):
                    oc[orow, pl.ds(obase + k * L, L)] = (
                        pb[j, pl.ds(off + k * L, L)])
            return carry

        lax.fori_loop(0, JGROUPS, g_body, 0)

    def fire_wb(oc, row, bb, wsem):
        pltpu.async_copy(oc, out_hbm.at[row, pl.ds(bb * EMBED, EMBED)], wsem)

    def wait_wb(wsem):
        pltpu.make_async_copy(oc0, out_hbm.at[0, pl.ds(0, EMBED)], wsem).wait()

    fire(pb0, pc0, gsem0, 0, 0)

    def row_body(si, carry):
        row = start + si

        def t_body(t, c2):
            bb0 = 2 * t
            not_first = (si > 0) | (t > 0)
            drain(pb0, gsem0)
            fire(pb1, pc1, gsem1, si, bb0 + 1)

            @pl.when(not_first)
            def _():
                wait_wb(wsem0)

            extract(pb0, oc0, si, bb0)
            fire_wb(oc0, row, bb0, wsem0)

            drain(pb1, gsem1)
            nsi = jnp.where(t < 3, si, si + 1)
            nbb = (bb0 + 2) % BLOCKS

            @pl.when((t < 3) | (si < n_rows - 1))
            def _():
                fire(pb0, pc0, gsem0, nsi, nbb)

            @pl.when(not_first)
            def _():
                wait_wb(wsem1)

            extract(pb1, oc1, si, bb0 + 1)
            fire_wb(oc1, row, bb0 + 1, wsem1)
            return c2

        lax.fori_loop(0, BLOCKS // 2, t_body, 0)
        return carry

    lax.fori_loop(0, n_rows, row_body, 0)
    wait_wb(wsem0)
    wait_wb(wsem1)


def _make_gather(seq, batch):
    return functools.partial(
        pl.kernel,
        out_type=jax.ShapeDtypeStruct((seq, batch // 2, CHUNK), jnp.float32),
        mesh=plsc.VectorSubcoreMesh(core_axis_name="c", subcore_axis_name="s"),
        scratch_types=[
            pltpu.VMEM((2 * GROUP, batch), jnp.int32),
            pltpu.VMEM((CHUNK,), jnp.int32),
            pltpu.VMEM((CHUNK,), jnp.int32),
            pltpu.VMEM((CHUNK, CHUNK), jnp.float32),
            pltpu.VMEM((CHUNK, CHUNK), jnp.float32),
            pltpu.VMEM((EMBED, CHUNK), jnp.float32),
            pltpu.VMEM((EMBED, CHUNK), jnp.float32),
            pltpu.SemaphoreType.DMA,
            pltpu.SemaphoreType.DMA,
            pltpu.SemaphoreType.DMA,
            pltpu.SemaphoreType.DMA,
        ],
        compiler_params=pltpu.CompilerParams(use_tc_tiling_on_sc=True),
    )(_gather_body)


def _pair_body(x_ref, o_ref):
    x = x_ref[...]
    eye = jnp.eye(x.shape[0], dtype=x.dtype)
    xT = jax.lax.dot_general(x, eye, (((0,), (0,)), ((), ())),
                             preferred_element_type=x.dtype)
    r3 = jnp.reshape(xT, (xT.shape[0] // 2, 2, xT.shape[1]))
    o_ref[...] = jnp.concatenate([r3[:, 0, :], r3[:, 1, :]], axis=1)


def _pair_table(word_embed):
    vocab, embed = word_embed.shape
    vb = 2048
    grid = (vocab + vb - 1) // vb
    return pl.pallas_call(
        _pair_body,
        grid=(grid,),
        in_specs=[pl.BlockSpec((embed, vb), lambda i: (0, i))],
        out_specs=pl.BlockSpec((vb // 2, 2 * embed), lambda i: (i, 0)),
        out_shape=jax.ShapeDtypeStruct((vocab // 2, 2 * embed), jnp.float32),
    )(word_embed.T)


def kernel(word_input, character_input, word_embed):
    batch, seq = word_input.shape
    vocab = word_embed.shape[0]
    tableP = _pair_table(word_embed)
    idx3 = jnp.reshape(word_input.T, (seq // GROUP, GROUP, batch))
    out = _make_gather(seq, batch)(tableP, idx3)
    return jnp.transpose(jnp.reshape(out, (seq, batch, EMBED)), (1, 0, 2))


# speedup vs baseline: 11.8163x; 1.0478x over previous
import functools

import jax
import jax.numpy as jnp
from jax import lax
from jax.experimental import pallas as pl
from jax.experimental.pallas import tpu as pltpu
from jax.experimental.pallas import tpu_sc as plsc

EMBED = 64
NUM_WORKERS = 32
CHUNK = 128
BLOCKS = 8
MAX_ROWS = 7


def _gather_body(table_hbm, idxT_hbm, out_hbm, idx_v, rows, gsem, osem):
    c = lax.axis_index("c")
    s = lax.axis_index("s")
    wid = s * 2 + c
    seq = idxT_hbm.shape[0]
    base_rows = seq // NUM_WORKERS
    n_extra = seq - base_rows * NUM_WORKERS
    start = jnp.where(wid < n_extra, wid * (base_rows + 1),
                      n_extra * (base_rows + 1) + (wid - n_extra) * base_rows)
    n_rows = jnp.where(wid < n_extra, base_rows + 1, base_rows)

    pltpu.sync_copy(idxT_hbm.at[pl.ds(start, base_rows)],
                    idx_v.at[pl.ds(0, base_rows)])

    @pl.when(n_rows > base_rows)
    def _():
        pltpu.sync_copy(idxT_hbm.at[pl.ds(start + base_rows, 1)],
                        idx_v.at[pl.ds(base_rows, 1)])

    for b in range(BLOCKS):
        pltpu.async_copy(table_hbm.at[idx_v.at[0, pl.ds(b * CHUNK, CHUNK)]],
                         rows.at[b], gsem.at[b])

    def row_body(si, carry):
        row = start + si
        for b in range(BLOCKS):
            pltpu.make_async_copy(
                table_hbm.at[idx_v.at[0, pl.ds(b * CHUNK, CHUNK)]],
                rows.at[b], gsem.at[b]).wait()
            pltpu.async_copy(rows.at[b], out_hbm.at[row, pl.ds(b * CHUNK, CHUNK)],
                             osem.at[b])

            @pl.when(si < n_rows - 1)
            def _():
                pltpu.make_async_copy(
                    rows.at[b], out_hbm.at[row, pl.ds(b * CHUNK, CHUNK)],
                    osem.at[b]).wait()
                pltpu.async_copy(
                    table_hbm.at[idx_v.at[si + 1, pl.ds(b * CHUNK, CHUNK)]],
                    rows.at[b], gsem.at[b])
        return carry

    lax.fori_loop(0, n_rows, row_body, 0)

    for b in range(BLOCKS):
        pltpu.make_async_copy(rows.at[b], out_hbm.at[0, pl.ds(0, CHUNK)],
                              osem.at[b]).wait()


def _make_gather(seq, batch):
    return functools.partial(
        pl.kernel,
        out_type=jax.ShapeDtypeStruct((seq, batch, EMBED), jnp.float32),
        mesh=plsc.VectorSubcoreMesh(core_axis_name="c", subcore_axis_name="s"),
        scratch_types=[
            pltpu.VMEM((MAX_ROWS, batch), jnp.int32),
            pltpu.VMEM((BLOCKS, CHUNK, EMBED), jnp.float32),
            pltpu.SemaphoreType.DMA((BLOCKS,)),
            pltpu.SemaphoreType.DMA((BLOCKS,)),
        ],
        compiler_params=pltpu.CompilerParams(use_tc_tiling_on_sc=False),
    )(_gather_body)


def kernel(word_input, character_input, word_embed):
    batch, seq = word_input.shape
    idxT = word_input.T
    out = _make_gather(seq, batch)(word_embed, idxT)
    return jnp.transpose(out, (1, 0, 2))
